# Initial kernel scaffold; baseline (speedup 1.0000x reference)
import jax
import jax.numpy as jnp
from jax.experimental import pallas as pl


def kernel(x, W_emb, Wq1, Wk1, Wv1, Wq2, Wk2, Wv2, W1, b1, W2, b2):
    raise NotImplementedError("write your pallas kernel here")


# speedup vs baseline: 3.6894x; 3.6894x over previous
import functools
import jax
import jax.numpy as jnp
from jax import lax
from jax.experimental import pallas as pl
from jax.experimental.pallas import tpu as pltpu
from jax.experimental.pallas import tpu_sc as plsc

L = 2048
D = 128
NH = 8
DK = D // NH
OUT = 1024

_NC = 2
_NS = 16
_NW = _NC * _NS
_BPW = L // _NW


def _pos_encoding():
    pos = jnp.arange(L, dtype=jnp.float32).reshape(-1, 1)
    div = jnp.power(10000.0, jnp.arange(0, D, 2, dtype=jnp.float32) / D)
    ang = pos / div
    p = jnp.zeros((L, D), dtype=jnp.float32)
    p = p.at[:, 0::2].set(jnp.sin(ang))
    p = p.at[:, 1::2].set(jnp.cos(ang))
    return p


_POS = _pos_encoding()

@functools.cache
def _sc_gather_fn():
    mesh = plsc.VectorSubcoreMesh(core_axis_name="c", subcore_axis_name="s")

    @functools.partial(
        pl.kernel,
        mesh=mesh,
        out_type=jax.ShapeDtypeStruct((L, D), jnp.float32),
        scratch_types=[
            pltpu.VMEM((_BPW,), jnp.int32),
            pltpu.VMEM((_BPW, D), jnp.float32),
            pltpu.SemaphoreType.DMA,
        ],
    )
    def _sc_gather(table_hbm, idx_hbm, out_hbm, idx_v, rows_v, sem):
        wid = lax.axis_index("s") * _NC + lax.axis_index("c")
        base = wid * _BPW
        pltpu.sync_copy(idx_hbm.at[pl.ds(base, _BPW)], idx_v)
        pltpu.async_copy(table_hbm.at[idx_v], rows_v, sem).wait()
        pltpu.sync_copy(rows_v, out_hbm.at[pl.ds(base, _BPW)])

    return _sc_gather


def _attn_math(xe, wq, wk, wv, o_ref):
    q = jnp.dot(xe, wq, preferred_element_type=jnp.float32)
    k = jnp.dot(xe, wk, preferred_element_type=jnp.float32)
    v = jnp.dot(xe, wv, preferred_element_type=jnp.float32)
    s = lax.dot_general(q, k, (((1,), (1,)), ((), ())),
                        preferred_element_type=jnp.float32)
    s = s * (1.0 / (DK ** 0.5))
    m = jnp.max(s, axis=-1, keepdims=True)
    e = jnp.exp(s - m)
    denom = jnp.sum(e, axis=-1, keepdims=True)
    av = jnp.dot(e, v, preferred_element_type=jnp.float32)
    o_ref[0] = av / denom


def _attn_body_pos(x_ref, pos_ref, wq_ref, wk_ref, wv_ref, o_ref):
    _attn_math(x_ref[...] + pos_ref[...], wq_ref[0], wk_ref[0],
               wv_ref[0], o_ref)


def _attn_body(x_ref, wq_ref, wk_ref, wv_ref, o_ref):
    _attn_math(x_ref[...], wq_ref[0], wk_ref[0], wv_ref[0], o_ref)


def _attn_layer(x, wq, wk, wv, pos=None):
    wq_h = wq.reshape(D, NH, DK).transpose(1, 0, 2)
    wk_h = wk.reshape(D, NH, DK).transpose(1, 0, 2)
    wv_h = wv.reshape(D, NH, DK).transpose(1, 0, 2)
    x_spec = pl.BlockSpec((L, D), lambda h: (0, 0))
    w_spec = pl.BlockSpec((1, D, DK), lambda h: (h, 0, 0))
    if pos is None:
        body = _attn_body
        in_specs = [x_spec, w_spec, w_spec, w_spec]
        args = (x, wq_h, wk_h, wv_h)
    else:
        body = _attn_body_pos
        in_specs = [x_spec, x_spec, w_spec, w_spec, w_spec]
        args = (x, pos, wq_h, wk_h, wv_h)
    o3 = pl.pallas_call(
        body,
        grid=(NH,),
        in_specs=in_specs,
        out_specs=pl.BlockSpec((1, L, DK), lambda h: (h, 0, 0)),
        out_shape=jax.ShapeDtypeStruct((NH, L, DK), jnp.float32),
    )(*args)
    return o3.transpose(1, 0, 2).reshape(L, D)


_KBLK = 32768
_KSTEPS = (L * D) // _KBLK


def _mlp_body(a_ref, w1_ref, b1_ref, w2_ref, b2_ref, out_ref, acc_ref):
    i = pl.program_id(0)

    @pl.when(i == 0)
    def _():
        acc_ref[...] = jnp.zeros_like(acc_ref)

    acc_ref[...] += jnp.dot(a_ref[...], w1_ref[...],
                            preferred_element_type=jnp.float32)

    @pl.when(i == _KSTEPS - 1)
    def _():
        h = jnp.maximum(acc_ref[...] + b1_ref[...], 0.0)
        z = jnp.dot(h, w2_ref[...], preferred_element_type=jnp.float32)
        z = z + b2_ref[...]
        p = jax.nn.softmax(z, axis=-1)
        out_ref[...] = jax.nn.softmax(p, axis=-1)


def _mlp_head(o2_flat, w1, b1, w2, b2):
    return pl.pallas_call(
        _mlp_body,
        grid=(_KSTEPS,),
        in_specs=[
            pl.BlockSpec((1, _KBLK), lambda i: (0, i)),
            pl.BlockSpec((_KBLK, D), lambda i: (i, 0)),
            pl.BlockSpec((1, D), lambda i: (0, 0)),
            pl.BlockSpec((D, OUT), lambda i: (0, 0)),
            pl.BlockSpec((1, OUT), lambda i: (0, 0)),
        ],
        out_specs=pl.BlockSpec((1, OUT), lambda i: (0, 0)),
        out_shape=jax.ShapeDtypeStruct((1, OUT), jnp.float32),
        scratch_shapes=[pltpu.VMEM((1, D), jnp.float32)],
    )(o2_flat, w1, b1, w2, b2)


def kernel(x, W_emb, Wq1, Wk1, Wv1, Wq2, Wk2, Wv2, W1, b1, W2, b2):
    x0 = x[0]
    e = _sc_gather_fn()(W_emb, x0)
    o = _attn_layer(e, Wq1, Wk1, Wv1, pos=_POS)
    o = _attn_layer(o, Wq2, Wk2, Wv2)
    out = _mlp_head(o.reshape(1, L * D), W1, b1.reshape(1, D), W2,
                    b2.reshape(1, OUT))
    return (out[0], jnp.float32(0.0))
